# baseline (speedup 1.0000x reference)
import math

import jax
import jax.numpy as jnp
from jax.experimental import pallas as pl
from jax.experimental.pallas import tpu as pltpu

F_DIM = 128
B_DIM = 128
BV_DIM = 6
HIDDEN = 64
HALF = HIDDEN // 2
B = 4
N = 4096
NUM_NODES = 16384

BLK = 256
NB = NUM_NODES // BLK
G = NB + (B - 1)
FBLK = 2048
VROWS = 72
LOGITS_SCALE = math.log2(math.e) / 8.0


def _bproj_kernel(b_pre_ref, bv_ref, Wb_ref, bb_ref, Wbv_ref, bbv_ref,
                  cbT_ref, bva_ref):
    cb = (
        jnp.dot(Wb_ref[...], b_pre_ref[0], preferred_element_type=jnp.float32)
        + bb_ref[...]
    )
    cbT_ref[0] = cb.T
    bv = (
        jnp.dot(Wbv_ref[...], bv_ref[0], preferred_element_type=jnp.float32)
        + bbv_ref[...]
    )
    bva_ref[0] = jnp.concatenate(
        [bv,
         jnp.ones((1, N), jnp.float32),
         jnp.zeros((VROWS - HIDDEN - 1, N), jnp.float32)],
        axis=0,
    )


def _fproj_kernel(Wf_ref, bf_ref, fpre_ref, out_ref):
    out_ref[...] = (
        jax.lax.dot_general(
            Wf_ref[...], fpre_ref[...],
            (((1,), (1,)), ((), ())),
            preferred_element_type=jnp.float32,
        )
        + bf_ref[...]
    ) * LOGITS_SCALE


def _attn_kernel(meta_ref, fproj_ref, cbT_ref, bva_ref, Wout_ref,
                 bout_ref, out_ref):
    g = pl.program_id(0)
    blk_j = meta_ref[1, g]
    seg_s = meta_ref[2, g]
    seg_e = meta_ref[3, g]

    fblk = fproj_ref[...]
    zero = jnp.zeros((HALF, BLK), jnp.float32)
    f_bd = jnp.concatenate(
        [jnp.concatenate([fblk[:HALF], zero], axis=1),
         jnp.concatenate([zero, fblk[HALF:]], axis=1)],
        axis=0,
    )
    logits = jnp.dot(cbT_ref[0], f_bd,
                     preferred_element_type=jnp.float32)
    p = jnp.exp2(logits)
    oa = jnp.dot(bva_ref[0], p,
                 preferred_element_type=jnp.float32)
    o = oa[:HIDDEN] / oa[HIDDEN:HIDDEN + 1]
    res = (
        jnp.dot(Wout_ref[:, :HIDDEN], o[:, :BLK],
                preferred_element_type=jnp.float32)
        + jnp.dot(Wout_ref[:, HIDDEN:], o[:, BLK:],
                  preferred_element_type=jnp.float32)
        + bout_ref[...]
    )

    node = blk_j * BLK + jax.lax.broadcasted_iota(jnp.int32, (BLK, 1), 0)
    mask = (node >= seg_s) & (node < seg_e)
    out_ref[...] = jnp.where(mask, res.T, out_ref[...])


def kernel(f_pre_in, f_pre_batch, b_pre_in, bv_in, Wf, bf, Wb, bb, Wbv, bbv,
           Wout, bout):
    fb = f_pre_batch.astype(jnp.int32)

    counts = jnp.sum(fb[None, :] == jnp.arange(B, dtype=jnp.int32)[:, None],
                     axis=1).astype(jnp.int32)
    ends = jnp.cumsum(counts).astype(jnp.int32)
    starts = ends - counts
    j0 = starts // BLK
    j1 = (ends - 1) // BLK
    nitems = jnp.where(counts > 0, j1 - j0 + 1, 0).astype(jnp.int32)
    cum = jnp.cumsum(nitems).astype(jnp.int32)
    total = cum[-1]

    g = jnp.arange(G, dtype=jnp.int32)
    bid_g = jnp.searchsorted(cum, g, side="right").astype(jnp.int32)
    bid_g = jnp.minimum(bid_g, B - 1)
    prev = jnp.where(bid_g > 0, cum[jnp.maximum(bid_g - 1, 0)], 0)
    blk_g = j0[bid_g] + (g - prev)
    live = g < total
    blk_g = jnp.where(live, blk_g, NB - 1)
    bid_g = jnp.where(live, bid_g, B - 1)
    s_g = jnp.where(live, starts[bid_g], 0)
    e_g = jnp.where(live, ends[bid_g], 0)
    meta = jnp.stack([bid_g, blk_g, s_g, e_g], axis=0)

    bb2 = bb.reshape(HIDDEN, 1)
    bbv2 = bbv.reshape(HIDDEN, 1)
    bf2 = bf.reshape(HIDDEN, 1)
    bout2 = bout.reshape(HIDDEN, 1)

    cbT, bva = pl.pallas_call(
        _bproj_kernel,
        grid=(B,),
        in_specs=[
            pl.BlockSpec((1, B_DIM, N), lambda i: (i, 0, 0)),
            pl.BlockSpec((1, BV_DIM, N), lambda i: (i, 0, 0)),
            pl.BlockSpec((HIDDEN, B_DIM), lambda i: (0, 0)),
            pl.BlockSpec((HIDDEN, 1), lambda i: (0, 0)),
            pl.BlockSpec((HIDDEN, BV_DIM), lambda i: (0, 0)),
            pl.BlockSpec((HIDDEN, 1), lambda i: (0, 0)),
        ],
        out_specs=[
            pl.BlockSpec((1, N, HIDDEN), lambda i: (i, 0, 0)),
            pl.BlockSpec((1, VROWS, N), lambda i: (i, 0, 0)),
        ],
        out_shape=[
            jax.ShapeDtypeStruct((B, N, HIDDEN), jnp.float32),
            jax.ShapeDtypeStruct((B, VROWS, N), jnp.float32),
        ],
        compiler_params=pltpu.CompilerParams(
            dimension_semantics=("arbitrary",)),
    )(b_pre_in, bv_in, Wb, bb2, Wbv, bbv2)

    fproj = pl.pallas_call(
        _fproj_kernel,
        grid=(NUM_NODES // FBLK,),
        in_specs=[
            pl.BlockSpec((HIDDEN, F_DIM), lambda i: (0, 0)),
            pl.BlockSpec((HIDDEN, 1), lambda i: (0, 0)),
            pl.BlockSpec((FBLK, F_DIM), lambda i: (i, 0)),
        ],
        out_specs=pl.BlockSpec((HIDDEN, FBLK), lambda i: (0, i)),
        out_shape=jax.ShapeDtypeStruct((HIDDEN, NUM_NODES), jnp.float32),
        compiler_params=pltpu.CompilerParams(
            dimension_semantics=("arbitrary",)),
    )(Wf, bf2, f_pre_in)

    grid_spec = pltpu.PrefetchScalarGridSpec(
        num_scalar_prefetch=1,
        grid=(G,),
        in_specs=[
            pl.BlockSpec((HIDDEN, BLK), lambda g, meta: (0, meta[1, g])),
            pl.BlockSpec((1, N, HIDDEN), lambda g, meta: (meta[0, g], 0, 0)),
            pl.BlockSpec((1, VROWS, N), lambda g, meta: (meta[0, g], 0, 0)),
            pl.BlockSpec((HIDDEN, 2 * HIDDEN), lambda g, meta: (0, 0)),
            pl.BlockSpec((HIDDEN, 1), lambda g, meta: (0, 0)),
        ],
        out_specs=pl.BlockSpec((BLK, HIDDEN), lambda g, meta: (meta[1, g], 0)),
    )

    out = pl.pallas_call(
        _attn_kernel,
        grid_spec=grid_spec,
        out_shape=jax.ShapeDtypeStruct((NUM_NODES, HIDDEN), jnp.float32),
        compiler_params=pltpu.CompilerParams(
            dimension_semantics=("arbitrary",)),
    )(meta, fproj, cbT, bva, Wout, bout2)

    return out


# speedup vs baseline: 1.2005x; 1.2005x over previous
import math

import jax
import jax.numpy as jnp
from jax.experimental import pallas as pl
from jax.experimental.pallas import tpu as pltpu

F_DIM = 128
B_DIM = 128
BV_DIM = 6
HIDDEN = 64
HALF = HIDDEN // 2
B = 4
N = 4096
NUM_NODES = 16384

BLK = 256
NB = NUM_NODES // BLK
G = NB + (B - 1)
GPAD = 128
FBLK = 2048
VROWS = 72
LOGITS_SCALE = math.log2(math.e) / 8.0


def _bproj_kernel(b_pre_ref, bv_ref, Wb_ref, bb_ref, Wbv_ref, bbv_ref, fb_ref,
                  cbT_ref, bva_ref, meta_ref):
    cb = (
        jnp.dot(Wb_ref[...], b_pre_ref[0], preferred_element_type=jnp.float32)
        + bb_ref[...]
    )
    cbT_ref[0] = cb.T
    bv = (
        jnp.dot(Wbv_ref[...], bv_ref[0], preferred_element_type=jnp.float32)
        + bbv_ref[...]
    )
    bva_ref[0] = jnp.concatenate(
        [bv,
         jnp.ones((1, N), jnp.float32),
         jnp.zeros((VROWS - HIDDEN - 1, N), jnp.float32)],
        axis=0,
    )

    @pl.when(pl.program_id(0) == B - 1)
    def _():
        fb2 = fb_ref[...]
        cnts = [jnp.sum(jnp.where(fb2 == b, 1, 0)) for b in range(B)]
        ends, acc = [], 0
        for b in range(B):
            acc = acc + cnts[b]
            ends.append(acc)
        starts = [ends[b] - cnts[b] for b in range(B)]
        j0 = [starts[b] // BLK for b in range(B)]
        j1 = [(ends[b] - 1) // BLK for b in range(B)]
        nit = [jnp.where(cnts[b] > 0, j1[b] - j0[b] + 1, 0) for b in range(B)]
        cum, acc2 = [], 0
        for b in range(B):
            acc2 = acc2 + nit[b]
            cum.append(acc2)
        total = cum[B - 1]
        gi = jax.lax.broadcasted_iota(jnp.int32, (1, GPAD), 1)
        bid = ((gi >= cum[0]).astype(jnp.int32)
               + (gi >= cum[1]).astype(jnp.int32)
               + (gi >= cum[2]).astype(jnp.int32))

        def sel4(vals):
            r = jnp.zeros((1, GPAD), jnp.int32) + vals[3]
            r = jnp.where(bid == 0, vals[0], r)
            r = jnp.where(bid == 1, vals[1], r)
            r = jnp.where(bid == 2, vals[2], r)
            return r

        prev = sel4([0, cum[0], cum[1], cum[2]])
        blk = sel4(j0) + (gi - prev)
        live = gi < total
        meta_ref[0:1] = jnp.where(live, bid, B - 1)
        meta_ref[1:2] = jnp.where(live, blk, NB - 1)
        meta_ref[2:3] = jnp.where(live, sel4(starts), 0)
        meta_ref[3:4] = jnp.where(live, sel4(ends), 0)
        meta_ref[4:8] = jnp.zeros((4, GPAD), jnp.int32)


def _fproj_kernel(Wf_ref, bf_ref, fpre_ref, out_ref):
    out_ref[...] = (
        jax.lax.dot_general(
            Wf_ref[...], fpre_ref[...],
            (((1,), (1,)), ((), ())),
            preferred_element_type=jnp.float32,
        )
        + bf_ref[...]
    ) * LOGITS_SCALE


def _attn_kernel(meta_ref, fproj_ref, cbT_ref, bva_ref, Wout_ref,
                 bout_ref, out_ref):
    g = pl.program_id(0)
    blk_j = meta_ref[1, g]
    seg_s = meta_ref[2, g]
    seg_e = meta_ref[3, g]

    fblk = fproj_ref[...]
    zero = jnp.zeros((HALF, BLK), jnp.float32)
    f_bd = jnp.concatenate(
        [jnp.concatenate([fblk[:HALF], zero], axis=1),
         jnp.concatenate([zero, fblk[HALF:]], axis=1)],
        axis=0,
    )
    logits = jnp.dot(cbT_ref[0], f_bd,
                     preferred_element_type=jnp.float32)
    p = jnp.exp2(logits)
    oa = jnp.dot(bva_ref[0], p,
                 preferred_element_type=jnp.float32)
    o = oa[:HIDDEN] / oa[HIDDEN:HIDDEN + 1]
    res = (
        jnp.dot(Wout_ref[:, :HIDDEN], o[:, :BLK],
                preferred_element_type=jnp.float32)
        + jnp.dot(Wout_ref[:, HIDDEN:], o[:, BLK:],
                  preferred_element_type=jnp.float32)
        + bout_ref[...]
    )

    node = blk_j * BLK + jax.lax.broadcasted_iota(jnp.int32, (1, BLK), 1)
    mask = (node >= seg_s) & (node < seg_e)
    out_ref[...] = jnp.where(mask, res, out_ref[...])


def kernel(f_pre_in, f_pre_batch, b_pre_in, bv_in, Wf, bf, Wb, bb, Wbv, bbv,
           Wout, bout):
    fb2d = f_pre_batch.astype(jnp.int32).reshape(128, NUM_NODES // 128)

    bb2 = bb.reshape(HIDDEN, 1)
    bbv2 = bbv.reshape(HIDDEN, 1)
    bf2 = bf.reshape(HIDDEN, 1)
    bout2 = bout.reshape(HIDDEN, 1)

    cbT, bva, meta = pl.pallas_call(
        _bproj_kernel,
        grid=(B,),
        in_specs=[
            pl.BlockSpec((1, B_DIM, N), lambda i: (i, 0, 0)),
            pl.BlockSpec((1, BV_DIM, N), lambda i: (i, 0, 0)),
            pl.BlockSpec((HIDDEN, B_DIM), lambda i: (0, 0)),
            pl.BlockSpec((HIDDEN, 1), lambda i: (0, 0)),
            pl.BlockSpec((HIDDEN, BV_DIM), lambda i: (0, 0)),
            pl.BlockSpec((HIDDEN, 1), lambda i: (0, 0)),
            pl.BlockSpec((128, NUM_NODES // 128), lambda i: (0, 0)),
        ],
        out_specs=[
            pl.BlockSpec((1, N, HIDDEN), lambda i: (i, 0, 0)),
            pl.BlockSpec((1, VROWS, N), lambda i: (i, 0, 0)),
            pl.BlockSpec((8, GPAD), lambda i: (0, 0)),
        ],
        out_shape=[
            jax.ShapeDtypeStruct((B, N, HIDDEN), jnp.float32),
            jax.ShapeDtypeStruct((B, VROWS, N), jnp.float32),
            jax.ShapeDtypeStruct((8, GPAD), jnp.int32),
        ],
        compiler_params=pltpu.CompilerParams(
            dimension_semantics=("arbitrary",)),
    )(b_pre_in, bv_in, Wb, bb2, Wbv, bbv2, fb2d)

    fproj = pl.pallas_call(
        _fproj_kernel,
        grid=(NUM_NODES // FBLK,),
        in_specs=[
            pl.BlockSpec((HIDDEN, F_DIM), lambda i: (0, 0)),
            pl.BlockSpec((HIDDEN, 1), lambda i: (0, 0)),
            pl.BlockSpec((FBLK, F_DIM), lambda i: (i, 0)),
        ],
        out_specs=pl.BlockSpec((HIDDEN, FBLK), lambda i: (0, i)),
        out_shape=jax.ShapeDtypeStruct((HIDDEN, NUM_NODES), jnp.float32),
        compiler_params=pltpu.CompilerParams(
            dimension_semantics=("arbitrary",)),
    )(Wf, bf2, f_pre_in)

    grid_spec = pltpu.PrefetchScalarGridSpec(
        num_scalar_prefetch=1,
        grid=(G,),
        in_specs=[
            pl.BlockSpec((HIDDEN, BLK), lambda g, meta: (0, meta[1, g])),
            pl.BlockSpec((1, N, HIDDEN), lambda g, meta: (meta[0, g], 0, 0)),
            pl.BlockSpec((1, VROWS, N), lambda g, meta: (meta[0, g], 0, 0)),
            pl.BlockSpec((HIDDEN, 2 * HIDDEN), lambda g, meta: (0, 0)),
            pl.BlockSpec((HIDDEN, 1), lambda g, meta: (0, 0)),
        ],
        out_specs=pl.BlockSpec((HIDDEN, BLK), lambda g, meta: (0, meta[1, g])),
    )

    out64 = pl.pallas_call(
        _attn_kernel,
        grid_spec=grid_spec,
        out_shape=jax.ShapeDtypeStruct((HIDDEN, NUM_NODES), jnp.float32),
        compiler_params=pltpu.CompilerParams(
            dimension_semantics=("arbitrary",)),
    )(meta, fproj, cbT, bva, Wout, bout2)

    return out64.T


# speedup vs baseline: 1.2393x; 1.0324x over previous
import math

import jax
import jax.numpy as jnp
from jax.experimental import pallas as pl
from jax.experimental.pallas import tpu as pltpu

F_DIM = 128
B_DIM = 128
BV_DIM = 6
HIDDEN = 64
HALF = HIDDEN // 2
B = 4
N = 4096
NUM_NODES = 16384

BLK = 512
NB = NUM_NODES // BLK
G = NB + (B - 1)
GPAD = 128
FBLK = 2048
VROWS = 72
LOGITS_SCALE = math.log2(math.e) / 8.0


def _bproj_kernel(b_pre_ref, bv_ref, Wb_ref, bb_ref, Wbv_ref, bbv_ref, fb_ref,
                  cbT_ref, bva_ref, meta_ref):
    cb = (
        jnp.dot(Wb_ref[...], b_pre_ref[0], preferred_element_type=jnp.float32)
        + bb_ref[...]
    )
    cbT_ref[0] = cb.T
    bv = (
        jnp.dot(Wbv_ref[...], bv_ref[0], preferred_element_type=jnp.float32)
        + bbv_ref[...]
    )
    bva_ref[0] = jnp.concatenate(
        [bv,
         jnp.ones((1, N), jnp.float32),
         jnp.zeros((VROWS - HIDDEN - 1, N), jnp.float32)],
        axis=0,
    )

    @pl.when(pl.program_id(0) == B - 1)
    def _():
        fb2 = fb_ref[...]
        cnts = [jnp.sum(jnp.where(fb2 == b, 1, 0)) for b in range(B)]
        ends, acc = [], 0
        for b in range(B):
            acc = acc + cnts[b]
            ends.append(acc)
        starts = [ends[b] - cnts[b] for b in range(B)]
        j0 = [starts[b] // BLK for b in range(B)]
        j1 = [(ends[b] - 1) // BLK for b in range(B)]
        nit = [jnp.where(cnts[b] > 0, j1[b] - j0[b] + 1, 0) for b in range(B)]
        cum, acc2 = [], 0
        for b in range(B):
            acc2 = acc2 + nit[b]
            cum.append(acc2)
        total = cum[B - 1]
        gi = jax.lax.broadcasted_iota(jnp.int32, (1, GPAD), 1)
        bid = ((gi >= cum[0]).astype(jnp.int32)
               + (gi >= cum[1]).astype(jnp.int32)
               + (gi >= cum[2]).astype(jnp.int32))

        def sel4(vals):
            r = jnp.zeros((1, GPAD), jnp.int32) + vals[3]
            r = jnp.where(bid == 0, vals[0], r)
            r = jnp.where(bid == 1, vals[1], r)
            r = jnp.where(bid == 2, vals[2], r)
            return r

        prev = sel4([0, cum[0], cum[1], cum[2]])
        blk = sel4(j0) + (gi - prev)
        live = gi < total
        meta_ref[0:1] = jnp.where(live, bid, B - 1)
        meta_ref[1:2] = jnp.where(live, blk, NB - 1)
        meta_ref[2:3] = jnp.where(live, sel4(starts), 0)
        meta_ref[3:4] = jnp.where(live, sel4(ends), 0)
        meta_ref[4:8] = jnp.zeros((4, GPAD), jnp.int32)


def _fproj_kernel(Wf_ref, bf_ref, fpre_ref, out_ref):
    out_ref[...] = (
        jax.lax.dot_general(
            Wf_ref[...], fpre_ref[...],
            (((1,), (1,)), ((), ())),
            preferred_element_type=jnp.float32,
        )
        + bf_ref[...]
    ) * LOGITS_SCALE


def _attn_kernel(meta_ref, fproj_ref, cbT_ref, bva_ref, Wout_ref,
                 bout_ref, out_ref):
    g = pl.program_id(0)
    blk_j = meta_ref[1, g]
    seg_s = meta_ref[2, g]
    seg_e = meta_ref[3, g]

    fblk = fproj_ref[...]
    zero = jnp.zeros((HALF, BLK), jnp.float32)
    f_bd = jnp.concatenate(
        [jnp.concatenate([fblk[:HALF], zero], axis=1),
         jnp.concatenate([zero, fblk[HALF:]], axis=1)],
        axis=0,
    )
    logits = jnp.dot(cbT_ref[0], f_bd,
                     preferred_element_type=jnp.float32)
    p = jnp.exp2(logits)
    oa = jnp.dot(bva_ref[0], p,
                 preferred_element_type=jnp.float32)
    o = oa[:HIDDEN] / oa[HIDDEN:HIDDEN + 1]
    res = (
        jnp.dot(Wout_ref[:, :HIDDEN], o[:, :BLK],
                preferred_element_type=jnp.float32)
        + jnp.dot(Wout_ref[:, HIDDEN:], o[:, BLK:],
                  preferred_element_type=jnp.float32)
        + bout_ref[...]
    )

    node = blk_j * BLK + jax.lax.broadcasted_iota(jnp.int32, (1, BLK), 1)
    mask = (node >= seg_s) & (node < seg_e)
    out_ref[...] = jnp.where(mask, res, out_ref[...])


def kernel(f_pre_in, f_pre_batch, b_pre_in, bv_in, Wf, bf, Wb, bb, Wbv, bbv,
           Wout, bout):
    fb2d = f_pre_batch.astype(jnp.int32).reshape(128, NUM_NODES // 128)

    bb2 = bb.reshape(HIDDEN, 1)
    bbv2 = bbv.reshape(HIDDEN, 1)
    bf2 = bf.reshape(HIDDEN, 1)
    bout2 = bout.reshape(HIDDEN, 1)

    cbT, bva, meta = pl.pallas_call(
        _bproj_kernel,
        grid=(B,),
        in_specs=[
            pl.BlockSpec((1, B_DIM, N), lambda i: (i, 0, 0)),
            pl.BlockSpec((1, BV_DIM, N), lambda i: (i, 0, 0)),
            pl.BlockSpec((HIDDEN, B_DIM), lambda i: (0, 0)),
            pl.BlockSpec((HIDDEN, 1), lambda i: (0, 0)),
            pl.BlockSpec((HIDDEN, BV_DIM), lambda i: (0, 0)),
            pl.BlockSpec((HIDDEN, 1), lambda i: (0, 0)),
            pl.BlockSpec((128, NUM_NODES // 128), lambda i: (0, 0)),
        ],
        out_specs=[
            pl.BlockSpec((1, N, HIDDEN), lambda i: (i, 0, 0)),
            pl.BlockSpec((1, VROWS, N), lambda i: (i, 0, 0)),
            pl.BlockSpec((8, GPAD), lambda i: (0, 0)),
        ],
        out_shape=[
            jax.ShapeDtypeStruct((B, N, HIDDEN), jnp.float32),
            jax.ShapeDtypeStruct((B, VROWS, N), jnp.float32),
            jax.ShapeDtypeStruct((8, GPAD), jnp.int32),
        ],
        compiler_params=pltpu.CompilerParams(
            dimension_semantics=("arbitrary",)),
    )(b_pre_in, bv_in, Wb, bb2, Wbv, bbv2, fb2d)

    fproj = pl.pallas_call(
        _fproj_kernel,
        grid=(NUM_NODES // FBLK,),
        in_specs=[
            pl.BlockSpec((HIDDEN, F_DIM), lambda i: (0, 0)),
            pl.BlockSpec((HIDDEN, 1), lambda i: (0, 0)),
            pl.BlockSpec((FBLK, F_DIM), lambda i: (i, 0)),
        ],
        out_specs=pl.BlockSpec((HIDDEN, FBLK), lambda i: (0, i)),
        out_shape=jax.ShapeDtypeStruct((HIDDEN, NUM_NODES), jnp.float32),
        compiler_params=pltpu.CompilerParams(
            dimension_semantics=("arbitrary",)),
    )(Wf, bf2, f_pre_in)

    grid_spec = pltpu.PrefetchScalarGridSpec(
        num_scalar_prefetch=1,
        grid=(G,),
        in_specs=[
            pl.BlockSpec((HIDDEN, BLK), lambda g, meta: (0, meta[1, g])),
            pl.BlockSpec((1, N, HIDDEN), lambda g, meta: (meta[0, g], 0, 0)),
            pl.BlockSpec((1, VROWS, N), lambda g, meta: (meta[0, g], 0, 0)),
            pl.BlockSpec((HIDDEN, 2 * HIDDEN), lambda g, meta: (0, 0)),
            pl.BlockSpec((HIDDEN, 1), lambda g, meta: (0, 0)),
        ],
        out_specs=pl.BlockSpec((HIDDEN, BLK), lambda g, meta: (0, meta[1, g])),
    )

    out64 = pl.pallas_call(
        _attn_kernel,
        grid_spec=grid_spec,
        out_shape=jax.ShapeDtypeStruct((HIDDEN, NUM_NODES), jnp.float32),
        compiler_params=pltpu.CompilerParams(
            dimension_semantics=("arbitrary",)),
    )(meta, fproj, cbT, bva, Wout, bout2)

    return out64.T


# speedup vs baseline: 1.2826x; 1.0349x over previous
import math

import jax
import jax.numpy as jnp
from jax.experimental import pallas as pl
from jax.experimental.pallas import tpu as pltpu

F_DIM = 128
B_DIM = 128
BV_DIM = 6
HIDDEN = 64
HALF = HIDDEN // 2
B = 4
N = 4096
NUM_NODES = 16384

BLK = 512
NB = NUM_NODES // BLK
G = NB + (B - 1)
GPAD = 128
FBLK = NUM_NODES // B
VROWS = 72
LOGITS_SCALE = math.log2(math.e) / 8.0


def _prep_kernel(b_pre_ref, bv_ref, Wb_ref, bb_ref, Wbv_ref, bbv_ref, fb_ref,
                 Wf_ref, bf_ref, fpre_ref,
                 cbT_ref, bva_ref, meta_ref, fproj_ref):
    fproj_ref[...] = (
        jax.lax.dot_general(
            Wf_ref[...], fpre_ref[...],
            (((1,), (1,)), ((), ())),
            preferred_element_type=jnp.float32,
        )
        + bf_ref[...]
    ) * LOGITS_SCALE

    cb = (
        jnp.dot(Wb_ref[...], b_pre_ref[0], preferred_element_type=jnp.float32)
        + bb_ref[...]
    )
    cbT_ref[0] = cb.T
    bv = (
        jnp.dot(Wbv_ref[...], bv_ref[0], preferred_element_type=jnp.float32)
        + bbv_ref[...]
    )
    bva_ref[0] = jnp.concatenate(
        [bv,
         jnp.ones((1, N), jnp.float32),
         jnp.zeros((VROWS - HIDDEN - 1, N), jnp.float32)],
        axis=0,
    )

    @pl.when(pl.program_id(0) == B - 1)
    def _():
        fb2 = fb_ref[...]
        cnts = [jnp.sum(jnp.where(fb2 == b, 1, 0)) for b in range(B)]
        ends, acc = [], 0
        for b in range(B):
            acc = acc + cnts[b]
            ends.append(acc)
        starts = [ends[b] - cnts[b] for b in range(B)]
        j0 = [starts[b] // BLK for b in range(B)]
        j1 = [(ends[b] - 1) // BLK for b in range(B)]
        nit = [jnp.where(cnts[b] > 0, j1[b] - j0[b] + 1, 0) for b in range(B)]
        cum, acc2 = [], 0
        for b in range(B):
            acc2 = acc2 + nit[b]
            cum.append(acc2)
        total = cum[B - 1]
        gi = jax.lax.broadcasted_iota(jnp.int32, (1, GPAD), 1)
        bid = ((gi >= cum[0]).astype(jnp.int32)
               + (gi >= cum[1]).astype(jnp.int32)
               + (gi >= cum[2]).astype(jnp.int32))

        def sel4(vals):
            r = jnp.zeros((1, GPAD), jnp.int32) + vals[3]
            r = jnp.where(bid == 0, vals[0], r)
            r = jnp.where(bid == 1, vals[1], r)
            r = jnp.where(bid == 2, vals[2], r)
            return r

        prev = sel4([0, cum[0], cum[1], cum[2]])
        blk = sel4(j0) + (gi - prev)
        live = gi < total
        meta_ref[0:1] = jnp.where(live, bid, B - 1)
        meta_ref[1:2] = jnp.where(live, blk, NB - 1)
        meta_ref[2:3] = jnp.where(live, sel4(starts), 0)
        meta_ref[3:4] = jnp.where(live, sel4(ends), 0)
        meta_ref[4:8] = jnp.zeros((4, GPAD), jnp.int32)


def _attn_kernel(meta_ref, fproj_ref, cbT_ref, bva_ref, Wout_ref,
                 bout_ref, out_ref):
    g = pl.program_id(0)
    blk_j = meta_ref[1, g]
    seg_s = meta_ref[2, g]
    seg_e = meta_ref[3, g]

    fblk = fproj_ref[...]
    zero = jnp.zeros((HALF, BLK), jnp.float32)
    f_bd = jnp.concatenate(
        [jnp.concatenate([fblk[:HALF], zero], axis=1),
         jnp.concatenate([zero, fblk[HALF:]], axis=1)],
        axis=0,
    )
    logits = jnp.dot(cbT_ref[0], f_bd,
                     preferred_element_type=jnp.float32)
    p = jnp.exp2(logits)
    oa = jnp.dot(bva_ref[0], p,
                 preferred_element_type=jnp.float32)
    o = oa[:HIDDEN] / oa[HIDDEN:HIDDEN + 1]
    res = (
        jnp.dot(Wout_ref[:, :HIDDEN], o[:, :BLK],
                preferred_element_type=jnp.float32)
        + jnp.dot(Wout_ref[:, HIDDEN:], o[:, BLK:],
                  preferred_element_type=jnp.float32)
        + bout_ref[...]
    )

    node = blk_j * BLK + jax.lax.broadcasted_iota(jnp.int32, (1, BLK), 1)
    mask = (node >= seg_s) & (node < seg_e)
    out_ref[...] = jnp.where(mask, res, out_ref[...])


def kernel(f_pre_in, f_pre_batch, b_pre_in, bv_in, Wf, bf, Wb, bb, Wbv, bbv,
           Wout, bout):
    fb2d = f_pre_batch.astype(jnp.int32).reshape(128, NUM_NODES // 128)

    bb2 = bb.reshape(HIDDEN, 1)
    bbv2 = bbv.reshape(HIDDEN, 1)
    bf2 = bf.reshape(HIDDEN, 1)
    bout2 = bout.reshape(HIDDEN, 1)

    cbT, bva, meta, fproj = pl.pallas_call(
        _prep_kernel,
        grid=(B,),
        in_specs=[
            pl.BlockSpec((1, B_DIM, N), lambda i: (i, 0, 0)),
            pl.BlockSpec((1, BV_DIM, N), lambda i: (i, 0, 0)),
            pl.BlockSpec((HIDDEN, B_DIM), lambda i: (0, 0)),
            pl.BlockSpec((HIDDEN, 1), lambda i: (0, 0)),
            pl.BlockSpec((HIDDEN, BV_DIM), lambda i: (0, 0)),
            pl.BlockSpec((HIDDEN, 1), lambda i: (0, 0)),
            pl.BlockSpec((128, NUM_NODES // 128), lambda i: (0, 0)),
            pl.BlockSpec((HIDDEN, F_DIM), lambda i: (0, 0)),
            pl.BlockSpec((HIDDEN, 1), lambda i: (0, 0)),
            pl.BlockSpec((FBLK, F_DIM), lambda i: (i, 0)),
        ],
        out_specs=[
            pl.BlockSpec((1, N, HIDDEN), lambda i: (i, 0, 0)),
            pl.BlockSpec((1, VROWS, N), lambda i: (i, 0, 0)),
            pl.BlockSpec((8, GPAD), lambda i: (0, 0)),
            pl.BlockSpec((HIDDEN, FBLK), lambda i: (0, i)),
        ],
        out_shape=[
            jax.ShapeDtypeStruct((B, N, HIDDEN), jnp.float32),
            jax.ShapeDtypeStruct((B, VROWS, N), jnp.float32),
            jax.ShapeDtypeStruct((8, GPAD), jnp.int32),
            jax.ShapeDtypeStruct((HIDDEN, NUM_NODES), jnp.float32),
        ],
        compiler_params=pltpu.CompilerParams(
            dimension_semantics=("arbitrary",)),
    )(b_pre_in, bv_in, Wb, bb2, Wbv, bbv2, fb2d, Wf, bf2, f_pre_in)

    grid_spec = pltpu.PrefetchScalarGridSpec(
        num_scalar_prefetch=1,
        grid=(G,),
        in_specs=[
            pl.BlockSpec((HIDDEN, BLK), lambda g, meta: (0, meta[1, g])),
            pl.BlockSpec((1, N, HIDDEN), lambda g, meta: (meta[0, g], 0, 0)),
            pl.BlockSpec((1, VROWS, N), lambda g, meta: (meta[0, g], 0, 0)),
            pl.BlockSpec((HIDDEN, 2 * HIDDEN), lambda g, meta: (0, 0)),
            pl.BlockSpec((HIDDEN, 1), lambda g, meta: (0, 0)),
        ],
        out_specs=pl.BlockSpec((HIDDEN, BLK), lambda g, meta: (0, meta[1, g])),
    )

    out64 = pl.pallas_call(
        _attn_kernel,
        grid_spec=grid_spec,
        out_shape=jax.ShapeDtypeStruct((HIDDEN, NUM_NODES), jnp.float32),
        compiler_params=pltpu.CompilerParams(
            dimension_semantics=("arbitrary",)),
    )(meta, fproj, cbT, bva, Wout, bout2)

    return out64.T


# speedup vs baseline: 1.3046x; 1.0172x over previous
import math

import jax
import jax.numpy as jnp
from jax.experimental import pallas as pl
from jax.experimental.pallas import tpu as pltpu

F_DIM = 128
B_DIM = 128
BV_DIM = 6
HIDDEN = 64
HALF = HIDDEN // 2
B = 4
N = 4096
NUM_NODES = 16384

BLK = 512
NB = NUM_NODES // BLK
G = NB + (B - 1)
GPAD = 128
FBLK = NUM_NODES // B
VROWS = 72
LOGITS_SCALE = math.log2(math.e) / 8.0


def _prep_kernel(b_pre_ref, bv_ref, Wb_ref, bb_ref, Wbv_ref, bbv_ref, fb_ref,
                 Wf_ref, bf_ref, fpre_ref,
                 cbT_ref, bva_ref, meta_ref, fproj_ref):
    fproj_ref[...] = ((
        jax.lax.dot_general(
            Wf_ref[...], fpre_ref[...],
            (((1,), (1,)), ((), ())),
            preferred_element_type=jnp.float32,
        )
        + bf_ref[...]
    ) * LOGITS_SCALE).astype(jnp.bfloat16)

    cb = (
        jnp.dot(Wb_ref[...], b_pre_ref[0], preferred_element_type=jnp.float32)
        + bb_ref[...]
    )
    cbT_ref[0] = cb.T.astype(jnp.bfloat16)
    bv = (
        jnp.dot(Wbv_ref[...], bv_ref[0], preferred_element_type=jnp.float32)
        + bbv_ref[...]
    )
    bva_ref[0] = jnp.concatenate(
        [bv,
         jnp.ones((1, N), jnp.float32),
         jnp.zeros((VROWS - HIDDEN - 1, N), jnp.float32)],
        axis=0,
    ).astype(jnp.bfloat16)

    @pl.when(pl.program_id(0) == B - 1)
    def _():
        fb2 = fb_ref[...]
        cnts = [jnp.sum(jnp.where(fb2 == b, 1, 0)) for b in range(B)]
        ends, acc = [], 0
        for b in range(B):
            acc = acc + cnts[b]
            ends.append(acc)
        starts = [ends[b] - cnts[b] for b in range(B)]
        j0 = [starts[b] // BLK for b in range(B)]
        j1 = [(ends[b] - 1) // BLK for b in range(B)]
        nit = [jnp.where(cnts[b] > 0, j1[b] - j0[b] + 1, 0) for b in range(B)]
        cum, acc2 = [], 0
        for b in range(B):
            acc2 = acc2 + nit[b]
            cum.append(acc2)
        total = cum[B - 1]
        gi = jax.lax.broadcasted_iota(jnp.int32, (1, GPAD), 1)
        bid = ((gi >= cum[0]).astype(jnp.int32)
               + (gi >= cum[1]).astype(jnp.int32)
               + (gi >= cum[2]).astype(jnp.int32))

        def sel4(vals):
            r = jnp.zeros((1, GPAD), jnp.int32) + vals[3]
            r = jnp.where(bid == 0, vals[0], r)
            r = jnp.where(bid == 1, vals[1], r)
            r = jnp.where(bid == 2, vals[2], r)
            return r

        prev = sel4([0, cum[0], cum[1], cum[2]])
        blk = sel4(j0) + (gi - prev)
        live = gi < total
        meta_ref[0:1] = jnp.where(live, bid, B - 1)
        meta_ref[1:2] = jnp.where(live, blk, NB - 1)
        meta_ref[2:3] = jnp.where(live, sel4(starts), 0)
        meta_ref[3:4] = jnp.where(live, sel4(ends), 0)
        meta_ref[4:8] = jnp.zeros((4, GPAD), jnp.int32)


def _attn_kernel(meta_ref, fproj_ref, cbT_ref, bva_ref, Wout_ref,
                 bout_ref, out_ref):
    g = pl.program_id(0)
    blk_j = meta_ref[1, g]
    seg_s = meta_ref[2, g]
    seg_e = meta_ref[3, g]

    fblk = fproj_ref[...]
    zero = jnp.zeros((HALF, BLK), jnp.bfloat16)
    f_bd = jnp.concatenate(
        [jnp.concatenate([fblk[:HALF], zero], axis=1),
         jnp.concatenate([zero, fblk[HALF:]], axis=1)],
        axis=0,
    )
    logits = jnp.dot(cbT_ref[0], f_bd,
                     preferred_element_type=jnp.float32)
    p = jnp.exp2(logits).astype(jnp.bfloat16)
    oa = jnp.dot(bva_ref[0], p,
                 preferred_element_type=jnp.float32)
    o = oa[:HIDDEN] / oa[HIDDEN:HIDDEN + 1]
    res = (
        jnp.dot(Wout_ref[:, :HIDDEN], o[:, :BLK],
                preferred_element_type=jnp.float32)
        + jnp.dot(Wout_ref[:, HIDDEN:], o[:, BLK:],
                  preferred_element_type=jnp.float32)
        + bout_ref[...]
    )

    node = blk_j * BLK + jax.lax.broadcasted_iota(jnp.int32, (1, BLK), 1)
    mask = (node >= seg_s) & (node < seg_e)
    out_ref[...] = jnp.where(mask, res, out_ref[...])


def kernel(f_pre_in, f_pre_batch, b_pre_in, bv_in, Wf, bf, Wb, bb, Wbv, bbv,
           Wout, bout):
    fb2d = f_pre_batch.astype(jnp.int32).reshape(128, NUM_NODES // 128)

    bb2 = bb.reshape(HIDDEN, 1)
    bbv2 = bbv.reshape(HIDDEN, 1)
    bf2 = bf.reshape(HIDDEN, 1)
    bout2 = bout.reshape(HIDDEN, 1)

    cbT, bva, meta, fproj = pl.pallas_call(
        _prep_kernel,
        grid=(B,),
        in_specs=[
            pl.BlockSpec((1, B_DIM, N), lambda i: (i, 0, 0)),
            pl.BlockSpec((1, BV_DIM, N), lambda i: (i, 0, 0)),
            pl.BlockSpec((HIDDEN, B_DIM), lambda i: (0, 0)),
            pl.BlockSpec((HIDDEN, 1), lambda i: (0, 0)),
            pl.BlockSpec((HIDDEN, BV_DIM), lambda i: (0, 0)),
            pl.BlockSpec((HIDDEN, 1), lambda i: (0, 0)),
            pl.BlockSpec((128, NUM_NODES // 128), lambda i: (0, 0)),
            pl.BlockSpec((HIDDEN, F_DIM), lambda i: (0, 0)),
            pl.BlockSpec((HIDDEN, 1), lambda i: (0, 0)),
            pl.BlockSpec((FBLK, F_DIM), lambda i: (i, 0)),
        ],
        out_specs=[
            pl.BlockSpec((1, N, HIDDEN), lambda i: (i, 0, 0)),
            pl.BlockSpec((1, VROWS, N), lambda i: (i, 0, 0)),
            pl.BlockSpec((8, GPAD), lambda i: (0, 0)),
            pl.BlockSpec((HIDDEN, FBLK), lambda i: (0, i)),
        ],
        out_shape=[
            jax.ShapeDtypeStruct((B, N, HIDDEN), jnp.bfloat16),
            jax.ShapeDtypeStruct((B, VROWS, N), jnp.bfloat16),
            jax.ShapeDtypeStruct((8, GPAD), jnp.int32),
            jax.ShapeDtypeStruct((HIDDEN, NUM_NODES), jnp.bfloat16),
        ],
        compiler_params=pltpu.CompilerParams(
            dimension_semantics=("arbitrary",)),
    )(b_pre_in, bv_in, Wb, bb2, Wbv, bbv2, fb2d, Wf, bf2, f_pre_in)

    grid_spec = pltpu.PrefetchScalarGridSpec(
        num_scalar_prefetch=1,
        grid=(G,),
        in_specs=[
            pl.BlockSpec((HIDDEN, BLK), lambda g, meta: (0, meta[1, g])),
            pl.BlockSpec((1, N, HIDDEN), lambda g, meta: (meta[0, g], 0, 0)),
            pl.BlockSpec((1, VROWS, N), lambda g, meta: (meta[0, g], 0, 0)),
            pl.BlockSpec((HIDDEN, 2 * HIDDEN), lambda g, meta: (0, 0)),
            pl.BlockSpec((HIDDEN, 1), lambda g, meta: (0, 0)),
        ],
        out_specs=pl.BlockSpec((HIDDEN, BLK), lambda g, meta: (0, meta[1, g])),
    )

    out64 = pl.pallas_call(
        _attn_kernel,
        grid_spec=grid_spec,
        out_shape=jax.ShapeDtypeStruct((HIDDEN, NUM_NODES), jnp.float32),
        compiler_params=pltpu.CompilerParams(
            dimension_semantics=("arbitrary",)),
    )(meta, fproj, cbT, bva, Wout, bout2)

    return out64.T


# speedup vs baseline: 1.3309x; 1.0202x over previous
import math

import jax
import jax.numpy as jnp
from jax.experimental import pallas as pl
from jax.experimental.pallas import tpu as pltpu

F_DIM = 128
B_DIM = 128
BV_DIM = 6
HIDDEN = 64
HALF = HIDDEN // 2
B = 4
N = 4096
NUM_NODES = 16384

BLK = 512
NB = NUM_NODES // BLK
G = NB + (B - 1)
FBLK = NUM_NODES // B
VROWS = 72
LOGITS_SCALE = math.log2(math.e) / 8.0


def _fused_kernel(b_pre_ref, bv_ref, fpre_ref, fb_ref, Wb_ref, bb_ref,
                  Wbv_ref, bbv_ref, Wf_ref, bf_ref, Wout_ref, bout_ref,
                  out_ref, cbT_s, bva_s, fproj_s, seg_s_ref):
    i = pl.program_id(0)

    @pl.when(i < B)
    def _prep():
        cb = (
            jnp.dot(Wb_ref[...], b_pre_ref[0],
                    preferred_element_type=jnp.float32)
            + bb_ref[...]
        )
        cbT_s[i] = cb.T.astype(jnp.bfloat16)
        bv = (
            jnp.dot(Wbv_ref[...], bv_ref[0],
                    preferred_element_type=jnp.float32)
            + bbv_ref[...]
        )
        bva_s[i] = jnp.concatenate(
            [bv,
             jnp.ones((1, N), jnp.float32),
             jnp.zeros((VROWS - HIDDEN - 1, N), jnp.float32)],
            axis=0,
        ).astype(jnp.bfloat16)
        fproj_s[:, pl.ds(i * FBLK, FBLK)] = ((
            jax.lax.dot_general(
                Wf_ref[...], fpre_ref[...],
                (((1,), (1,)), ((), ())),
                preferred_element_type=jnp.float32,
            )
            + bf_ref[...]
        ) * LOGITS_SCALE).astype(jnp.bfloat16)

        @pl.when(i == B - 1)
        def _meta():
            fb2 = fb_ref[...]
            cnts = [jnp.sum(jnp.where(fb2 == b, 1, 0)) for b in range(B)]
            acc = 0
            ends = []
            for b in range(B):
                acc = acc + cnts[b]
                ends.append(acc)
            starts = [ends[b] - cnts[b] for b in range(B)]
            j0 = [starts[b] // BLK for b in range(B)]
            j1 = [(ends[b] - 1) // BLK for b in range(B)]
            nit = [jnp.where(cnts[b] > 0, j1[b] - j0[b] + 1, 0)
                   for b in range(B)]
            acc2 = 0
            for b in range(B):
                acc2 = acc2 + nit[b]
                seg_s_ref[b] = acc2
                seg_s_ref[4 + b] = j0[b]
                seg_s_ref[8 + b] = starts[b]
                seg_s_ref[12 + b] = ends[b]

    @pl.when(i >= B)
    def _attn():
        g = i - B
        cum = [seg_s_ref[b] for b in range(B)]
        bid = ((g >= cum[0]).astype(jnp.int32)
               + (g >= cum[1]).astype(jnp.int32)
               + (g >= cum[2]).astype(jnp.int32))

        def sel4(base):
            r = seg_s_ref[base + 3]
            r = jnp.where(bid == 0, seg_s_ref[base + 0], r)
            r = jnp.where(bid == 1, seg_s_ref[base + 1], r)
            r = jnp.where(bid == 2, seg_s_ref[base + 2], r)
            return r

        prev = jnp.where(bid == 0, 0,
                         jnp.where(bid == 1, cum[0],
                                   jnp.where(bid == 2, cum[1], cum[2])))
        blk = sel4(4) + (g - prev)
        live = g < cum[3]
        blk = jnp.where(live, blk, NB - 1)
        bid = jnp.where(live, bid, B - 1)
        seg_lo = jnp.where(live, sel4(8), 0)
        seg_hi = jnp.where(live, sel4(12), 0)
        off = blk * BLK

        fblk = fproj_s[:, pl.ds(off, BLK)]
        zero = jnp.zeros((HALF, BLK), jnp.bfloat16)
        f_bd = jnp.concatenate(
            [jnp.concatenate([fblk[:HALF], zero], axis=1),
             jnp.concatenate([zero, fblk[HALF:]], axis=1)],
            axis=0,
        )
        logits = jnp.dot(cbT_s[bid], f_bd,
                         preferred_element_type=jnp.float32)
        p = jnp.exp2(logits).astype(jnp.bfloat16)
        oa = jnp.dot(bva_s[bid], p,
                     preferred_element_type=jnp.float32)
        o = oa[:HIDDEN] / oa[HIDDEN:HIDDEN + 1]
        res = (
            jnp.dot(Wout_ref[:, :HIDDEN], o[:, :BLK],
                    preferred_element_type=jnp.float32)
            + jnp.dot(Wout_ref[:, HIDDEN:], o[:, BLK:],
                      preferred_element_type=jnp.float32)
            + bout_ref[...]
        )

        node = off + jax.lax.broadcasted_iota(jnp.int32, (1, BLK), 1)
        mask = (node >= seg_lo) & (node < seg_hi)
        cur = out_ref[:, pl.ds(off, BLK)]
        out_ref[:, pl.ds(off, BLK)] = jnp.where(mask, res, cur)


def kernel(f_pre_in, f_pre_batch, b_pre_in, bv_in, Wf, bf, Wb, bb, Wbv, bbv,
           Wout, bout):
    fb2d = f_pre_batch.astype(jnp.int32).reshape(128, NUM_NODES // 128)

    bb2 = bb.reshape(HIDDEN, 1)
    bbv2 = bbv.reshape(HIDDEN, 1)
    bf2 = bf.reshape(HIDDEN, 1)
    bout2 = bout.reshape(HIDDEN, 1)

    out64 = pl.pallas_call(
        _fused_kernel,
        grid=(B + G,),
        in_specs=[
            pl.BlockSpec((1, B_DIM, N), lambda i: (jnp.minimum(i, B - 1), 0, 0)),
            pl.BlockSpec((1, BV_DIM, N), lambda i: (jnp.minimum(i, B - 1), 0, 0)),
            pl.BlockSpec((FBLK, F_DIM), lambda i: (jnp.minimum(i, B - 1), 0)),
            pl.BlockSpec((128, NUM_NODES // 128), lambda i: (0, 0)),
            pl.BlockSpec((HIDDEN, B_DIM), lambda i: (0, 0)),
            pl.BlockSpec((HIDDEN, 1), lambda i: (0, 0)),
            pl.BlockSpec((HIDDEN, BV_DIM), lambda i: (0, 0)),
            pl.BlockSpec((HIDDEN, 1), lambda i: (0, 0)),
            pl.BlockSpec((HIDDEN, F_DIM), lambda i: (0, 0)),
            pl.BlockSpec((HIDDEN, 1), lambda i: (0, 0)),
            pl.BlockSpec((HIDDEN, 2 * HIDDEN), lambda i: (0, 0)),
            pl.BlockSpec((HIDDEN, 1), lambda i: (0, 0)),
        ],
        out_specs=pl.BlockSpec((HIDDEN, NUM_NODES), lambda i: (0, 0)),
        out_shape=jax.ShapeDtypeStruct((HIDDEN, NUM_NODES), jnp.float32),
        scratch_shapes=[
            pltpu.VMEM((B, N, HIDDEN), jnp.bfloat16),
            pltpu.VMEM((B, VROWS, N), jnp.bfloat16),
            pltpu.VMEM((HIDDEN, NUM_NODES), jnp.bfloat16),
            pltpu.SMEM((16,), jnp.int32),
        ],
        compiler_params=pltpu.CompilerParams(
            dimension_semantics=("arbitrary",)),
    )(b_pre_in, bv_in, f_pre_in, fb2d, Wb, bb2, Wbv, bbv2, Wf, bf2, Wout,
      bout2)

    return out64.T
